# Initial kernel scaffold; baseline (speedup 1.0000x reference)
import jax
import jax.numpy as jnp
from jax.experimental import pallas as pl


def kernel(x_entity, x_transaction, edge_index_gat, edge_index_sage, W_gat, att_src, att_dst, bias_gat, W_l, b_l, W_r, ln_s_ent, ln_b_ent, ln_s_tx, ln_b_tx):
    raise NotImplementedError("write your pallas kernel here")


# speedup vs baseline: 26.8259x; 26.8259x over previous
import jax
import jax.numpy as jnp
from jax import lax
from jax.experimental import pallas as pl
from jax.experimental.pallas import tpu as pltpu
from jax.experimental.pallas import tpu_sc as plsc

N = 10000
D = 128
H = 4
E = 160000

NC = 2
NS = 16
NW = NC * NS
LANES = 16

NP = 10240
CPT = NP // NS
K = 256
NCHUNK = E // K
ITERS = -(-NCHUNK // NW)
ZR = 32

_mesh = plsc.VectorSubcoreMesh(core_axis_name="c", subcore_axis_name="s")


def _fill_vmem_2d(ref, rows, cols, value):
    v = jnp.full((LANES,), value, jnp.float32)
    per_row = cols // LANES

    def body(i, _):
        r = i // per_row
        c = (i % per_row) * LANES
        ref[r, pl.ds(c, LANES)] = v
        return 0

    lax.fori_loop(0, rows * per_row, body, 0)


def _fill_vmem_1d(ref, n, value):
    v = jnp.full((LANES,), value, jnp.float32)

    def body(i, _):
        ref[pl.ds(i * LANES, LANES)] = v
        return 0

    lax.fori_loop(0, n // LANES, body, 0)


def _pre_body(x_ref, wg_ref, asrc_ref, adst_ref, wr_ref,
              h_ref, ssrc_ref, sdst_ref, xr_ref):
    xb = x_ref[...]
    for hh in range(H):
        h_h = jnp.dot(xb, wg_ref[:, hh * D:(hh + 1) * D],
                      preferred_element_type=jnp.float32)
        h_ref[hh] = h_h
        ssrc_ref[hh] = jnp.sum(h_h * asrc_ref[hh, :][None, :], axis=1,
                               keepdims=True)
        sdst_ref[hh] = jnp.sum(h_h * adst_ref[hh, :][None, :], axis=1,
                               keepdims=True)
    xr_ref[...] = jnp.dot(xb, wr_ref[...], preferred_element_type=jnp.float32)


def _precompute(x_entity, W_gat, att_src, att_dst, W_r):
    blk = 1000
    return pl.pallas_call(
        _pre_body,
        grid=(N // blk,),
        in_specs=[
            pl.BlockSpec((blk, D), lambda i: (i, 0)),
            pl.BlockSpec((D, H * D), lambda i: (0, 0)),
            pl.BlockSpec((H, D), lambda i: (0, 0)),
            pl.BlockSpec((H, D), lambda i: (0, 0)),
            pl.BlockSpec((D, D), lambda i: (0, 0)),
        ],
        out_specs=[
            pl.BlockSpec((H, blk, D), lambda i: (0, i, 0)),
            pl.BlockSpec((H, blk, 1), lambda i: (0, i, 0)),
            pl.BlockSpec((H, blk, 1), lambda i: (0, i, 0)),
            pl.BlockSpec((blk, D), lambda i: (i, 0)),
        ],
        out_shape=[
            jax.ShapeDtypeStruct((H, N, D), jnp.float32),
            jax.ShapeDtypeStruct((H, N, 1), jnp.float32),
            jax.ShapeDtypeStruct((H, N, 1), jnp.float32),
            jax.ShapeDtypeStruct((N, D), jnp.float32),
        ],
    )(x_entity, W_gat, att_src, att_dst, W_r)


def _gat_sc_body(h0, h1, h2, h3, ssrc_hbm, sdst_hbm, src_hbm, dst_hbm,
                 den_out, acc_out,
                 sval, dval, srcv, dstv, exbuf, adj, dadj, rows, zacc,
                 gsem, ssem, dsem,
                 den_sh, acc_sh):
    cid = lax.axis_index("c")
    sid = lax.axis_index("s")
    wid = sid * NC + cid
    hrefs = (h0, h1, h2, h3)

    _fill_vmem_2d(zacc, ZR, D, 0.0)
    for z in range(CPT // ZR):
        pltpu.sync_copy(zacc, acc_sh.at[pl.ds(sid * CPT + z * ZR, ZR)])
    for hh in range(H):
        for z in range(CPT // D):
            pltpu.sync_copy(zacc.at[0],
                            den_sh.at[pl.ds(hh * NP + sid * CPT + z * D, D)])
    plsc.subcore_barrier()

    for hh in range(H):
        def chunk(t, _, hh=hh):
            c = wid + t * NW

            @pl.when(c < NCHUNK)
            def _():
                base = c * K
                pltpu.sync_copy(src_hbm.at[pl.ds(base, K)], srcv)
                pltpu.sync_copy(dst_hbm.at[pl.ds(base, K)], dstv)
                gather = pltpu.async_copy(hrefs[hh].at[srcv], rows, gsem)
                def mkadj(i, _):
                    adj[pl.ds(i * LANES, LANES)] = (
                        srcv[pl.ds(i * LANES, LANES)] + hh * N)
                    adj[pl.ds(K + i * LANES, LANES)] = (
                        dstv[pl.ds(i * LANES, LANES)] + hh * N)
                    return 0

                lax.fori_loop(0, K // LANES, mkadj, 0)
                gs = pltpu.async_copy(ssrc_hbm.at[adj.at[pl.ds(0, K)]],
                                      sval, ssem)
                gd = pltpu.async_copy(sdst_hbm.at[adj.at[pl.ds(K, K)]],
                                      dval, dsem)
                gs.wait()
                gd.wait()

                def vec(i, _):
                    al = (sval[pl.ds(i * LANES, LANES)]
                          + dval[pl.ds(i * LANES, LANES)])
                    al = jnp.where(al > 0, al, al * 0.2)
                    exbuf[pl.ds(i * LANES, LANES)] = jnp.exp(al)
                    return 0

                lax.fori_loop(0, K // LANES, vec, 0)
                gather.wait()

                def scale(i, _):
                    w16 = exbuf[pl.ds(i * LANES, LANES)]
                    for ee in range(LANES):
                        e = i * LANES + ee
                        w = w16[ee]
                        for j in range(D // LANES):
                            rows[e, pl.ds(j * LANES, LANES)] = (
                                rows[e, pl.ds(j * LANES, LANES)] * w)
                    return 0

                lax.fori_loop(0, K // LANES, scale, 0)
                pltpu.sync_copy(rows, acc_sh.at[dstv], add=True)

                def mkden(i, _):
                    dadj[pl.ds(i * LANES, LANES)] = (
                        dstv[pl.ds(i * LANES, LANES)] + hh * NP)
                    return 0

                lax.fori_loop(0, K // LANES, mkden, 0)
                pltpu.sync_copy(exbuf, den_sh.at[dadj], add=True)
            return 0

        lax.fori_loop(0, ITERS, chunk, 0)
        plsc.subcore_barrier()
        pltpu.sync_copy(
            acc_sh.at[pl.ds(sid * CPT, CPT)],
            acc_out.at[pl.ds((cid * H + hh) * NP + sid * CPT, CPT)])
        if hh < H - 1:
            for z in range(CPT // ZR):
                pltpu.sync_copy(zacc,
                                acc_sh.at[pl.ds(sid * CPT + z * ZR, ZR)])
        plsc.subcore_barrier()

    for hh in range(H):
        pltpu.sync_copy(
            den_sh.at[pl.ds(hh * NP + sid * CPT, CPT)],
            den_out.at[pl.ds((cid * H + hh) * NP + sid * CPT, CPT)])


def _gat_edges(h_heads, ssrc, sdst, src, dst):
    kfn = pl.kernel(
        _gat_sc_body,
        out_type=[
            jax.ShapeDtypeStruct((NC * H * NP,), jnp.float32),
            jax.ShapeDtypeStruct((NC * H * NP, D), jnp.float32),
        ],
        mesh=_mesh,
        scratch_types=[
            pltpu.VMEM((K,), jnp.float32),
            pltpu.VMEM((K,), jnp.float32),
            pltpu.VMEM((K,), jnp.int32),
            pltpu.VMEM((K,), jnp.int32),
            pltpu.VMEM((K,), jnp.float32),
            pltpu.VMEM((2 * K,), jnp.int32),
            pltpu.VMEM((K,), jnp.int32),
            pltpu.VMEM((K, D), jnp.float32),
            pltpu.VMEM((ZR, D), jnp.float32),
            pltpu.SemaphoreType.DMA,
            pltpu.SemaphoreType.DMA,
            pltpu.SemaphoreType.DMA,
            pltpu.VMEM_SHARED((H * NP,), jnp.float32),
            pltpu.VMEM_SHARED((NP, D), jnp.float32),
        ],
    )
    den_flat, acc_flat = kfn(h_heads[0], h_heads[1], h_heads[2], h_heads[3],
                             ssrc.reshape(H * N), sdst.reshape(H * N),
                             src, dst)
    return den_flat.reshape(NC, H, NP, 1), acc_flat.reshape(NC, H, NP, D)


def _sage_sc_body(xtx_hbm, src_hbm, dst_hbm,
                  s_out, cnt_out,
                  srcv, dstv, rows, ones, zacc, gsem,
                  acc_sh, cnt_sh):
    cid = lax.axis_index("c")
    sid = lax.axis_index("s")
    wid = sid * NC + cid

    _fill_vmem_2d(zacc, ZR, D, 0.0)
    _fill_vmem_1d(ones, K, 1.0)
    for z in range(CPT // ZR):
        pltpu.sync_copy(zacc, acc_sh.at[pl.ds(sid * CPT + z * ZR, ZR)])
    for z in range(CPT // D):
        pltpu.sync_copy(zacc.at[0], cnt_sh.at[pl.ds(sid * CPT + z * D, D)])
    plsc.subcore_barrier()

    def chunk(t, _):
        c = wid + t * NW

        @pl.when(c < NCHUNK)
        def _():
            base = c * K
            pltpu.sync_copy(src_hbm.at[pl.ds(base, K)], srcv)
            pltpu.sync_copy(dst_hbm.at[pl.ds(base, K)], dstv)
            pltpu.async_copy(xtx_hbm.at[srcv], rows, gsem).wait()
            pltpu.sync_copy(rows, acc_sh.at[dstv], add=True)
            pltpu.sync_copy(ones, cnt_sh.at[dstv], add=True)
        return 0

    lax.fori_loop(0, ITERS, chunk, 0)
    plsc.subcore_barrier()
    pltpu.sync_copy(acc_sh.at[pl.ds(sid * CPT, CPT)],
                    s_out.at[pl.ds(cid * NP + sid * CPT, CPT)])
    pltpu.sync_copy(cnt_sh.at[pl.ds(sid * CPT, CPT)],
                    cnt_out.at[pl.ds(cid * NP + sid * CPT, CPT)])


def _sage_edges(x_transaction, src, dst):
    kfn = pl.kernel(
        _sage_sc_body,
        out_type=[
            jax.ShapeDtypeStruct((NC * NP, D), jnp.float32),
            jax.ShapeDtypeStruct((NC * NP,), jnp.float32),
        ],
        mesh=_mesh,
        scratch_types=[
            pltpu.VMEM((K,), jnp.int32),
            pltpu.VMEM((K,), jnp.int32),
            pltpu.VMEM((K, D), jnp.float32),
            pltpu.VMEM((K,), jnp.float32),
            pltpu.VMEM((ZR, D), jnp.float32),
            pltpu.SemaphoreType.DMA,
            pltpu.VMEM_SHARED((NP, D), jnp.float32),
            pltpu.VMEM_SHARED((NP,), jnp.float32),
        ],
    )
    s_flat, cnt_flat = kfn(x_transaction, src, dst)
    return s_flat.reshape(NC, NP, D), cnt_flat.reshape(NC, NP, 1)


def _ln_elu(x, s, b):
    mu = jnp.mean(x, axis=-1, keepdims=True)
    var = jnp.mean((x - mu) ** 2, axis=-1, keepdims=True)
    y = (x - mu) / jnp.sqrt(var + 1e-5) * s + b
    return jnp.where(y > 0, y, jnp.exp(y) - 1.0)


def _combine_body(acc_ref, den_ref, ss_ref, cnt_ref, xr_ref, xe_ref, xt_ref,
                  wl_ref, bl_ref, bg_ref, lse_ref, lbe_ref, lst_ref, lbt_ref,
                  ent_ref, tx_ref):
    acc = acc_ref[0] + acc_ref[1]
    den = den_ref[0] + den_ref[1]
    gat = jnp.mean(acc / (den + 1e-16), axis=0) + bg_ref[0, :]

    st = ss_ref[0] + ss_ref[1]
    cnt = jnp.clip(cnt_ref[0] + cnt_ref[1], 1.0, None)
    mean = st / cnt
    sage = (jnp.dot(mean, wl_ref[...], preferred_element_type=jnp.float32)
            + bl_ref[0, :] + xr_ref[...])

    ent = gat + sage + xe_ref[...]
    ent_ref[...] = _ln_elu(ent, lse_ref[0, :], lbe_ref[0, :])
    tx_ref[...] = _ln_elu(xt_ref[...], lst_ref[0, :], lbt_ref[0, :])


def _combine(acc_p, den_p, sage_p, cnt_p, xr, x_entity, x_transaction,
             W_l, b_l, bias_gat, ln_s_ent, ln_b_ent, ln_s_tx, ln_b_tx):
    blk = 1000
    row2d = lambda v: v.reshape(1, D)
    return pl.pallas_call(
        _combine_body,
        grid=(N // blk,),
        in_specs=[
            pl.BlockSpec((NC, H, blk, D), lambda i: (0, 0, i, 0)),
            pl.BlockSpec((NC, H, blk, 1), lambda i: (0, 0, i, 0)),
            pl.BlockSpec((NC, blk, D), lambda i: (0, i, 0)),
            pl.BlockSpec((NC, blk, 1), lambda i: (0, i, 0)),
            pl.BlockSpec((blk, D), lambda i: (i, 0)),
            pl.BlockSpec((blk, D), lambda i: (i, 0)),
            pl.BlockSpec((blk, D), lambda i: (i, 0)),
            pl.BlockSpec((D, D), lambda i: (0, 0)),
            pl.BlockSpec((1, D), lambda i: (0, 0)),
            pl.BlockSpec((1, D), lambda i: (0, 0)),
            pl.BlockSpec((1, D), lambda i: (0, 0)),
            pl.BlockSpec((1, D), lambda i: (0, 0)),
            pl.BlockSpec((1, D), lambda i: (0, 0)),
            pl.BlockSpec((1, D), lambda i: (0, 0)),
        ],
        out_specs=[
            pl.BlockSpec((blk, D), lambda i: (i, 0)),
            pl.BlockSpec((blk, D), lambda i: (i, 0)),
        ],
        out_shape=[
            jax.ShapeDtypeStruct((N, D), jnp.float32),
            jax.ShapeDtypeStruct((N, D), jnp.float32),
        ],
    )(acc_p, den_p, sage_p, cnt_p, xr, x_entity, x_transaction,
      W_l, row2d(b_l), row2d(bias_gat), row2d(ln_s_ent), row2d(ln_b_ent),
      row2d(ln_s_tx), row2d(ln_b_tx))


def kernel(x_entity, x_transaction, edge_index_gat, edge_index_sage,
           W_gat, att_src, att_dst, bias_gat, W_l, b_l, W_r,
           ln_s_ent, ln_b_ent, ln_s_tx, ln_b_tx):
    src_g = edge_index_gat[0].astype(jnp.int32)
    dst_g = edge_index_gat[1].astype(jnp.int32)
    src_s = edge_index_sage[0].astype(jnp.int32)
    dst_s = edge_index_sage[1].astype(jnp.int32)

    h_heads, ssrc, sdst, xr = _precompute(x_entity, W_gat, att_src,
                                          att_dst, W_r)
    den_p, acc_p = _gat_edges(h_heads, ssrc, sdst, src_g, dst_g)
    sage_p, cnt_p = _sage_edges(x_transaction, src_s, dst_s)
    ent, tx = _combine(acc_p, den_p, sage_p, cnt_p, xr,
                       x_entity, x_transaction, W_l, b_l, bias_gat,
                       ln_s_ent, ln_b_ent, ln_s_tx, ln_b_tx)
    return ent, tx
